# baseline (speedup 1.0000x reference)
import functools

import jax
import jax.numpy as jnp
from jax.experimental import pallas as pl
from jax.experimental.pallas import tpu as pltpu


def _dmpnn_kernel(ml_ref, f_ref, map_ref, a2b_ref, atom_ref, wi_ref, wh_ref,
                  woa_ref, wob_ref, bo_ref, comp_ref, mask_ref,
                  *, nb, na, maxb, comp_dim, mols):
    f32 = jnp.float32
    iota_bb = jax.lax.broadcasted_iota(jnp.int32, (nb, nb), 0)
    iota_ba = jax.lax.broadcasted_iota(jnp.int32, (nb, na), 0)
    dn_t = (((0,), (0,)), ((), ()))

    for i in range(mols):
        f = f_ref[i]
        inp = jnp.dot(f, wi_ref[...], preferred_element_type=f32)
        msg = jnp.maximum(inp, 0.0)

        m = map_ref[i]
        adj_t = jnp.zeros((nb, nb), dtype=f32)
        for k in range(maxb):
            adj_t = adj_t + (m[k][None, :] == iota_bb).astype(f32)

        for _ in range(2):
            agg = jax.lax.dot_general(adj_t, msg, dn_t,
                                      preferred_element_type=f32)
            msg = jnp.maximum(inp + jnp.dot(agg, wh_ref[...],
                                            preferred_element_type=f32), 0.0)

        a = a2b_ref[i]
        adj_at = jnp.zeros((nb, na), dtype=f32)
        for k in range(maxb):
            adj_at = adj_at + (a[k][None, :] == iota_ba).astype(f32)
        atom_msg = jax.lax.dot_general(adj_at, msg, dn_t,
                                       preferred_element_type=f32)

        hidden = jnp.dot(atom_ref[i], woa_ref[...],
                         preferred_element_type=f32)
        hidden = hidden + jnp.dot(atom_msg, wob_ref[...],
                                  preferred_element_type=f32)
        hidden = jnp.maximum(hidden + bo_ref[...], 0.0)
        comp_ref[pl.ds(i * na, na), :] = hidden

    mask_rows = (jax.lax.broadcasted_iota(jnp.int32, (mols, 1, comp_dim), 2)
                 < ml_ref[0]).astype(f32)
    mask_ref[...] = mask_rows


def kernel(atom_features, f_ini_atoms_bonds, atom_to_incoming_bonds, mapping,
           global_features, W_i, W_h, W_o, b_o, mol_len):
    B, NA, AF = atom_features.shape
    _, NB, AFBF = f_ini_atoms_bonds.shape
    H = W_i.shape[1]
    MAXB = mapping.shape[2]
    comp_dim = max(NA, H)
    assert comp_dim == H

    ml = jnp.asarray(mol_len, jnp.int32).reshape(1)
    mapping = mapping.astype(jnp.int32).transpose(0, 2, 1)
    a2b = atom_to_incoming_bonds.astype(jnp.int32).transpose(0, 2, 1)
    W_oa = W_o[:AF]
    W_ob = W_o[AF:]
    b_o2 = b_o.reshape(1, H)

    MOLS = 8
    body = functools.partial(_dmpnn_kernel, nb=NB, na=NA, maxb=MAXB,
                             comp_dim=comp_dim, mols=MOLS)

    comp, c_mask = pl.pallas_call(
        body,
        grid=(B // MOLS,),
        compiler_params=pltpu.CompilerParams(
            dimension_semantics=("parallel",),
            allow_input_fusion=[True] * 10,
            fuse_transposed_lhs_in_matmul=True),
        in_specs=[
            pl.BlockSpec(memory_space=pltpu.SMEM),
            pl.BlockSpec((MOLS, NB, AFBF), lambda b: (b, 0, 0)),
            pl.BlockSpec((MOLS, MAXB, NB), lambda b: (b, 0, 0)),
            pl.BlockSpec((MOLS, MAXB, NA), lambda b: (b, 0, 0)),
            pl.BlockSpec((MOLS, NA, AF), lambda b: (b, 0, 0)),
            pl.BlockSpec((AFBF, H), lambda b: (0, 0)),
            pl.BlockSpec((H, H), lambda b: (0, 0)),
            pl.BlockSpec((AF, H), lambda b: (0, 0)),
            pl.BlockSpec((H, H), lambda b: (0, 0)),
            pl.BlockSpec((1, H), lambda b: (0, 0)),
        ],
        out_specs=[
            pl.BlockSpec((MOLS * NA, comp_dim), lambda b: (b, 0)),
            pl.BlockSpec((MOLS, 1, comp_dim), lambda b: (b, 0, 0)),
        ],
        out_shape=[
            jax.ShapeDtypeStruct((B * NA, comp_dim), jnp.float32),
            jax.ShapeDtypeStruct((B, 1, comp_dim), jnp.float32),
        ],
    )(ml, f_ini_atoms_bonds, mapping, a2b, atom_features, W_i, W_h, W_oa,
      W_ob, b_o2)
    return comp, c_mask.reshape(B, comp_dim)


# speedup vs baseline: 1.1309x; 1.1309x over previous
import functools

import jax
import jax.numpy as jnp
from jax.experimental import pallas as pl
from jax.experimental.pallas import tpu as pltpu


def _dmpnn_kernel(ml_ref, f_ref, map_ref, a2b_ref, atom_ref, wi_ref, wh_ref,
                  woa_ref, wob_ref, bo_ref, comp_ref, mask_ref,
                  *, nb, na, maxb, comp_dim, mols):
    f32 = jnp.float32
    iota_b = jax.lax.broadcasted_iota(jnp.int32, (nb, nb), 1)
    iota_a = jax.lax.broadcasted_iota(jnp.int32, (na, nb), 1)

    for i in range(mols):
        f = f_ref[i]
        inp = jnp.dot(f, wi_ref[...], preferred_element_type=f32)
        msg = jnp.maximum(inp, 0.0)

        m = map_ref[i]
        adj = jnp.zeros((nb, nb), dtype=f32)
        for k in range(maxb):
            adj = adj + (m[:, k][:, None] == iota_b).astype(f32)

        for _ in range(2):
            agg = jnp.dot(adj, msg, preferred_element_type=f32)
            msg = jnp.maximum(inp + jnp.dot(agg, wh_ref[...],
                                            preferred_element_type=f32), 0.0)

        a = a2b_ref[i]
        adj_a = jnp.zeros((na, nb), dtype=f32)
        for k in range(maxb):
            adj_a = adj_a + (a[:, k][:, None] == iota_a).astype(f32)
        atom_msg = jnp.dot(adj_a, msg, preferred_element_type=f32)

        hidden = jnp.dot(atom_ref[i], woa_ref[...],
                         preferred_element_type=f32)
        hidden = hidden + jnp.dot(atom_msg, wob_ref[...],
                                  preferred_element_type=f32)
        hidden = jnp.maximum(hidden + bo_ref[...], 0.0)
        comp_ref[pl.ds(i * na, na), :] = hidden

    mask_rows = (jax.lax.broadcasted_iota(jnp.int32, (mols, 1, comp_dim), 2)
                 < ml_ref[0]).astype(f32)
    mask_ref[...] = mask_rows


def kernel(atom_features, f_ini_atoms_bonds, atom_to_incoming_bonds, mapping,
           global_features, W_i, W_h, W_o, b_o, mol_len):
    B, NA, AF = atom_features.shape
    _, NB, AFBF = f_ini_atoms_bonds.shape
    H = W_i.shape[1]
    MAXB = mapping.shape[2]
    comp_dim = max(NA, H)
    assert comp_dim == H

    ml = jnp.asarray(mol_len, jnp.int32).reshape(1)
    mapping = mapping.astype(jnp.int32)
    a2b = atom_to_incoming_bonds.astype(jnp.int32)
    W_oa = W_o[:AF]
    W_ob = W_o[AF:]
    b_o2 = b_o.reshape(1, H)

    MOLS = 16
    body = functools.partial(_dmpnn_kernel, nb=NB, na=NA, maxb=MAXB,
                             comp_dim=comp_dim, mols=MOLS)

    comp, c_mask = pl.pallas_call(
        body,
        grid=(B // MOLS,),
        compiler_params=pltpu.CompilerParams(
            dimension_semantics=("parallel",),
            allow_input_fusion=[True] * 10),
        in_specs=[
            pl.BlockSpec(memory_space=pltpu.SMEM),
            pl.BlockSpec((MOLS, NB, AFBF), lambda b: (b, 0, 0)),
            pl.BlockSpec((MOLS, NB, MAXB), lambda b: (b, 0, 0)),
            pl.BlockSpec((MOLS, NA, MAXB), lambda b: (b, 0, 0)),
            pl.BlockSpec((MOLS, NA, AF), lambda b: (b, 0, 0)),
            pl.BlockSpec((AFBF, H), lambda b: (0, 0)),
            pl.BlockSpec((H, H), lambda b: (0, 0)),
            pl.BlockSpec((AF, H), lambda b: (0, 0)),
            pl.BlockSpec((H, H), lambda b: (0, 0)),
            pl.BlockSpec((1, H), lambda b: (0, 0)),
        ],
        out_specs=[
            pl.BlockSpec((MOLS * NA, comp_dim), lambda b: (b, 0)),
            pl.BlockSpec((MOLS, 1, comp_dim), lambda b: (b, 0, 0)),
        ],
        out_shape=[
            jax.ShapeDtypeStruct((B * NA, comp_dim), jnp.float32),
            jax.ShapeDtypeStruct((B, 1, comp_dim), jnp.float32),
        ],
    )(ml, f_ini_atoms_bonds, mapping, a2b, atom_features, W_i, W_h, W_oa,
      W_ob, b_o2)
    return comp, c_mask.reshape(B, comp_dim)
